# Initial kernel scaffold; baseline (speedup 1.0000x reference)
import jax
import jax.numpy as jnp
from jax.experimental import pallas as pl


def kernel(input_ids, vectors, w, token_mapping, head_W, head_b):
    raise NotImplementedError("write your pallas kernel here")


# speedup vs baseline: 1.0134x; 1.0134x over previous
import functools

import jax
import jax.numpy as jnp
from jax import lax
from jax.experimental import pallas as pl
from jax.experimental.pallas import tpu as pltpu
from jax.experimental.pallas import tpu_sc as plsc

V = 1000000
D = 64
B = 4096
L = 200
OUT = 2

NC = 2
NS = 16
NW = NC * NS
RPW = B // NW
TPW = RPW * L
LC0, LC1 = 104, 96


def _sc_pool_body(ids_hbm, vec_hbm, w_hbm, tm_hbm, pooled_hbm,
                  ids_v, wv_v, idse_v, rowbuf_v, pooled_v,
                  sem_w, sem_tm, sem_r0, sem_r1):
    c = lax.axis_index("c")
    s = lax.axis_index("s")
    wid = s * NC + c
    tok0 = wid * TPW

    pltpu.sync_copy(ids_hbm.at[pl.ds(tok0, TPW)], ids_v)

    def fire_wtm(k, _):
        sl = pl.ds(k * 128, 128)
        idx = ids_v.at[sl]
        pltpu.async_copy(w_hbm.at[idx], wv_v.at[sl], sem_w)
        pltpu.async_copy(tm_hbm.at[idx], idse_v.at[sl], sem_tm)
        return 0
    lax.fori_loop(0, TPW // 128, fire_wtm, 0)
    pltpu.make_async_copy(w_hbm.at[pl.ds(0, TPW)], wv_v.at[pl.ds(0, TPW)],
                          sem_w).wait()
    pltpu.make_async_copy(tm_hbm.at[pl.ds(0, TPW)], idse_v, sem_tm).wait()

    def sig(k, _):
        sl = pl.ds(k * 16, 16)
        x = wv_v[sl]
        idv = ids_v[sl]
        sg = 1.0 / (1.0 + jnp.exp(-x))
        wv_v[sl] = jnp.where(idv != 0, sg, jnp.zeros_like(sg))
        return 0
    lax.fori_loop(0, TPW // 16, sig, 0)

    sems = (sem_r0, sem_r1)

    def fire_row(r, slot):
        off = r * L
        pltpu.async_copy(vec_hbm.at[idse_v.at[pl.ds(off, LC0)]],
                         rowbuf_v.at[slot, pl.ds(0, LC0)], sems[slot])
        pltpu.async_copy(vec_hbm.at[idse_v.at[pl.ds(off + LC0, LC1)]],
                         rowbuf_v.at[slot, pl.ds(LC0, LC1)], sems[slot])

    def drain_row(slot):
        pltpu.make_async_copy(vec_hbm.at[pl.ds(0, L)], rowbuf_v.at[slot],
                              sems[slot]).wait()

    fire_row(0, 0)
    fire_row(1, 1)

    def process(r, slot):
        drain_row(slot)
        base = r * L

        def fma_block(k, nt, carry):
            a0, a1, a2, a3 = carry
            wc = wv_v[pl.ds(base + k * 16, 16)]
            for t2 in range(nt):
                sc = wc[t2]
                t = k * 16 + t2
                a0 = a0 + sc * rowbuf_v[slot, t, pl.ds(0, 16)]
                a1 = a1 + sc * rowbuf_v[slot, t, pl.ds(16, 16)]
                a2 = a2 + sc * rowbuf_v[slot, t, pl.ds(32, 16)]
                a3 = a3 + sc * rowbuf_v[slot, t, pl.ds(48, 16)]
            return (a0, a1, a2, a3)

        z = jnp.zeros((16,), jnp.float32)
        carry = lax.fori_loop(0, L // 16, lambda k, cy: fma_block(k, 16, cy),
                              (z, z, z, z))
        a0, a1, a2, a3 = fma_block(L // 16, L % 16, carry)
        pooled_v[r, pl.ds(0, 16)] = a0
        pooled_v[r, pl.ds(16, 16)] = a1
        pooled_v[r, pl.ds(32, 16)] = a2
        pooled_v[r, pl.ds(48, 16)] = a3

    def row_group(g, _):
        for b in range(2):
            r = g * 2 + b
            process(r, b)

            @pl.when(r + 2 < RPW)
            def _():
                fire_row(r + 2, b)
        return 0
    lax.fori_loop(0, RPW // 2, row_group, 0)

    row0 = wid * RPW
    pltpu.sync_copy(pooled_v, pooled_hbm.at[pl.ds(row0, RPW)])


@jax.jit
def _sc_pool(ids_flat, vectors, w, token_mapping):
    mesh = plsc.VectorSubcoreMesh(core_axis_name="c", subcore_axis_name="s")
    fn = pl.kernel(
        _sc_pool_body,
        mesh=mesh,
        compiler_params=pltpu.CompilerParams(use_tc_tiling_on_sc=False),
        out_type=jax.ShapeDtypeStruct((B, D), jnp.float32),
        scratch_types=[
            pltpu.VMEM((TPW,), jnp.int32),
            pltpu.VMEM((TPW + 16,), jnp.float32),
            pltpu.VMEM((TPW,), jnp.int32),
            pltpu.VMEM((2, L, D), jnp.float32),
            pltpu.VMEM((RPW, D), jnp.float32),
            pltpu.SemaphoreType.DMA,
            pltpu.SemaphoreType.DMA,
            pltpu.SemaphoreType.DMA,
            pltpu.SemaphoreType.DMA,
        ],
    )
    return fn(ids_flat, vectors, w, token_mapping)


def _head_body(pooled_ref, ids_ref, w_ref, b_ref, logits_ref, enc_ref):
    raw = pooled_ref[...]
    ln = jnp.sum((ids_ref[...] != 0).astype(jnp.float32), axis=1,
                 keepdims=True) + 1e-16
    pooled = raw / ln
    ss = jnp.sum(pooled * pooled, axis=1, keepdims=True)
    nrm = jnp.sqrt(ss)
    enc = pooled / jnp.maximum(nrm, 1e-12)
    enc_ref[...] = enc
    logits_ref[...] = (
        jnp.dot(enc, w_ref[...], preferred_element_type=jnp.float32)
        + b_ref[...]
    )


@jax.jit
def _head(pooled, input_ids, head_W, head_b2d):
    blk = 512
    grid = B // blk
    return pl.pallas_call(
        _head_body,
        grid=(grid,),
        in_specs=[
            pl.BlockSpec((blk, D), lambda i: (i, 0)),
            pl.BlockSpec((blk, L), lambda i: (i, 0)),
            pl.BlockSpec((D, OUT), lambda i: (0, 0)),
            pl.BlockSpec((1, OUT), lambda i: (0, 0)),
        ],
        out_specs=[
            pl.BlockSpec((blk, OUT), lambda i: (i, 0)),
            pl.BlockSpec((blk, D), lambda i: (i, 0)),
        ],
        out_shape=[
            jax.ShapeDtypeStruct((B, OUT), jnp.float32),
            jax.ShapeDtypeStruct((B, D), jnp.float32),
        ],
    )(pooled, input_ids, head_W, head_b2d)


def kernel(input_ids, vectors, w, token_mapping, head_W, head_b):
    ids_flat = input_ids.reshape(-1)
    pooled = _sc_pool(ids_flat, vectors, w, token_mapping)
    logits, enc = _head(pooled, input_ids, head_W, head_b.reshape(1, OUT))
    return (logits, enc)
